# baseline (speedup 1.0000x reference)
import functools

import jax
import jax.numpy as jnp
from jax import lax
from jax.experimental import pallas as pl
from jax.experimental.pallas import tpu as pltpu
from jax.experimental.pallas import tpu_sc as plsc

_B = 1024
_T = 200
_D = 64
_CHUNK = 100
_NCHUNKS = _B * _T // _CHUNK
_VPC = _CHUNK * _D // 16


def _make_sc_kernel():
    info = plsc.get_sparse_core_info()
    nc, ns = info.num_cores, info.num_subcores
    nw = nc * ns
    chunks_per_w = _NCHUNKS // nw

    mesh = plsc.VectorSubcoreMesh(core_axis_name="c", subcore_axis_name="s")

    @functools.partial(
        pl.kernel,
        mesh=mesh,
        out_type=jax.ShapeDtypeStruct((_NCHUNKS, _CHUNK, _D), jnp.float32),
        scratch_types=[
            pltpu.VMEM((chunks_per_w, _CHUNK), jnp.int32),
            pltpu.VMEM((_T, _D), jnp.float32),
            pltpu.VMEM((_CHUNK, _D), jnp.float32),
            pltpu.SemaphoreType.DMA,
        ],
        compiler_params=pltpu.CompilerParams(use_tc_tiling_on_sc=False),
    )
    def k(idx_hbm, word_hbm, pos_hbm, out_hbm, idx_v, pos_v, buf, sem):
        wid = lax.axis_index("s") * nc + lax.axis_index("c")
        base_chunk = wid * chunks_per_w
        pltpu.sync_copy(idx_hbm.at[pl.ds(base_chunk, chunks_per_w)], idx_v)
        pltpu.sync_copy(pos_hbm, pos_v)

        def body(c, _):
            g = base_chunk + c
            par = lax.rem(g, _T // _CHUNK)
            t0 = par * _CHUNK
            pltpu.async_copy(word_hbm.at[idx_v.at[c]], buf, sem).wait()

            def add_pos(r, _):
                for d in range(_D // 16):
                    sl = pl.ds(d * 16, 16)
                    buf[r, sl] += pos_v[t0 + r, sl]
                return ()

            lax.fori_loop(0, _CHUNK, add_pos, (), unroll=4)
            pltpu.sync_copy(buf, out_hbm.at[g])
            return ()

        lax.fori_loop(0, chunks_per_w, body, (), unroll=False)

    return k


_sc_kernel = _make_sc_kernel()


@jax.jit
def kernel(x, word_table, pos_table):
    idx = x.reshape(_NCHUNKS, _CHUNK).astype(jnp.int32)
    out = _sc_kernel(idx, word_table, pos_table)
    return out.reshape(_B, _T, _D)


# speedup vs baseline: 1.1214x; 1.1214x over previous
import functools

import jax
import jax.numpy as jnp
from jax import lax
from jax.experimental import pallas as pl
from jax.experimental.pallas import tpu as pltpu
from jax.experimental.pallas import tpu_sc as plsc

_B = 1024
_T = 200
_D = 64
_CHUNK = 100
_NCHUNKS = _B * _T // _CHUNK
_SEQ_CHUNKS = _T // _CHUNK


def _make_sc_kernel():
    info = plsc.get_sparse_core_info()
    nc, ns = info.num_cores, info.num_subcores
    nw = nc * ns
    cpw = _NCHUNKS // nw

    mesh = plsc.VectorSubcoreMesh(core_axis_name="c", subcore_axis_name="s")

    @functools.partial(
        pl.kernel,
        mesh=mesh,
        out_type=jax.ShapeDtypeStruct((_NCHUNKS, _CHUNK, _D), jnp.float32),
        scratch_types=[
            pltpu.VMEM((cpw, _CHUNK), jnp.int32),
            pltpu.VMEM((_T, _D), jnp.float32),
            pltpu.VMEM((_CHUNK, _D), jnp.float32),
            pltpu.VMEM((_CHUNK, _D), jnp.float32),
            pltpu.SemaphoreType.DMA,
            pltpu.SemaphoreType.DMA,
            pltpu.SemaphoreType.DMA,
            pltpu.SemaphoreType.DMA,
        ],
        compiler_params=pltpu.CompilerParams(use_tc_tiling_on_sc=False),
    )
    def k(idx_hbm, word_hbm, pos_hbm, out_hbm,
          idx_v, pos_v, b0, b1, gs0, gs1, ws0, ws1):
        bufs, gs, ws = (b0, b1), (gs0, gs1), (ws0, ws1)
        wid = lax.axis_index("s") * nc + lax.axis_index("c")
        base = wid * cpw
        pltpu.sync_copy(idx_hbm.at[pl.ds(base, cpw)], idx_v)
        pltpu.sync_copy(pos_hbm, pos_v)

        def gather(c, b):
            pltpu.async_copy(word_hbm.at[idx_v.at[c]], bufs[b], gs[b])

        def wait_gather(b):
            pltpu.make_async_copy(bufs[b], out_hbm.at[0], gs[b]).wait()

        def write(c, b):
            pltpu.async_copy(bufs[b], out_hbm.at[base + c], ws[b])

        def wait_write(b):
            pltpu.make_async_copy(bufs[b], out_hbm.at[0], ws[b]).wait()

        def add_pos(c, b):
            buf = bufs[b]
            t0 = lax.rem(base + c, _SEQ_CHUNKS) * _CHUNK

            def row(r, _):
                for d in range(_D // 16):
                    sl = pl.ds(d * 16, 16)
                    plsc.addupdate(buf.at[r, sl], pos_v[t0 + r, sl])
                return ()

            lax.fori_loop(0, _CHUNK, row, (), unroll=4)

        gather(0, 0)
        wait_gather(0)
        add_pos(0, 0)
        gather(1, 1)
        write(0, 0)

        @pl.loop(1, cpw - 1, step=2)
        def _pair(c0):
            for j in range(2):
                c = c0 + j
                b = (1 + j) % 2
                wait_gather(b)
                add_pos(c, b)
                wait_write(1 - b)
                gather(c + 1, 1 - b)
                write(c, b)

        wait_gather(1)
        add_pos(cpw - 1, 1)
        write(cpw - 1, 1)
        wait_write(0)
        wait_write(1)

    return k


_sc_kernel = _make_sc_kernel()


@jax.jit
def kernel(x, word_table, pos_table):
    idx = x.reshape(_NCHUNKS, _CHUNK).astype(jnp.int32)
    out = _sc_kernel(idx, word_table, pos_table)
    return out.reshape(_B, _T, _D)
